# Initial kernel scaffold; baseline (speedup 1.0000x reference)
import jax
import jax.numpy as jnp
from jax.experimental import pallas as pl


def kernel(s, edge_index, edge_attr, batch, ln_gamma, ln_beta, We, be, Wm, bm, Wn, bn):
    raise NotImplementedError("write your pallas kernel here")


# speedup vs baseline: 2.9025x; 2.9025x over previous
import functools

import jax
import jax.numpy as jnp
from jax import lax
from jax.experimental import pallas as pl
from jax.experimental.pallas import tpu as pltpu
from jax.experimental.pallas import tpu_sc as plsc

G = 64
NW = 32
CHUNK = 1024
IDX_W = 128


def _ln_layer_body(D, eps, s_ref, br_ref, bc_ref, gam_ref, bet_ref, w2_ref,
                   s_out_ref, psrc_ref, pdst_ref):
    n = s_ref.shape[0]
    iota_row = lax.broadcasted_iota(jnp.int32, (1, G), 1)
    iota_col = lax.broadcasted_iota(jnp.int32, (G, 1), 0)
    onehot = (br_ref[...] == iota_row).astype(jnp.float32)
    onehot_t = (iota_col == bc_ref[...]).astype(jnp.float32)
    ones_n = jnp.ones((n, 1), jnp.float32)
    cnt_g = lax.dot_general(onehot_t, ones_n, (((1,), (0,)), ((), ())))
    cnt_d = jnp.maximum(cnt_g * jnp.float32(D), 1.0)
    s = s_ref[...]
    t = jnp.sum(s, axis=1, keepdims=True)
    mean_g = lax.dot_general(onehot_t, t, (((1,), (0,)), ((), ()))) / cnt_d
    mean_n = lax.dot_general(onehot, mean_g, (((1,), (0,)), ((), ())))
    diff = s - mean_n
    sq = jnp.sum(diff * diff, axis=1, keepdims=True)
    var_g = lax.dot_general(onehot_t, sq, (((1,), (0,)), ((), ()))) / cnt_d
    var_n = lax.dot_general(onehot, var_g, (((1,), (0,)), ((), ())))
    s = diff / jnp.sqrt(var_n + eps) * gam_ref[...] + bet_ref[...]
    s_out_ref[...] = s
    p = lax.dot_general(s, w2_ref[...], (((1,), (0,)), ((), ())),
                        preferred_element_type=jnp.float32)
    psrc_ref[...] = p[:, :16]
    pdst_ref[...] = p[:, 16:]


def _ln_layer(s, batch_r, batch_c, gamma, beta, w2_i):
    n, d = s.shape
    body = functools.partial(_ln_layer_body, d, 1e-5)
    return pl.pallas_call(
        body,
        out_shape=[
            jax.ShapeDtypeStruct((n, d), jnp.float32),
            jax.ShapeDtypeStruct((n, 16), jnp.float32),
            jax.ShapeDtypeStruct((n, 16), jnp.float32),
        ],
    )(s, batch_r, batch_c, gamma.reshape(1, d), beta.reshape(1, d), w2_i)


def _edge_mm_body(e_ref, w_ref, b_ref, o_ref):
    o_ref[...] = lax.dot_general(
        e_ref[...], w_ref[...], (((1,), (0,)), ((), ())),
        preferred_element_type=jnp.float32) + b_ref[...]


def _edge_mm(e, w, b):
    ep, ed = e.shape
    be_blk = 8192
    return pl.pallas_call(
        _edge_mm_body,
        grid=(ep // be_blk,),
        in_specs=[
            pl.BlockSpec((be_blk, ed), lambda i: (i, 0)),
            pl.BlockSpec((ed, ed), lambda i: (0, 0)),
            pl.BlockSpec((1, ed), lambda i: (0, 0)),
        ],
        out_specs=pl.BlockSpec((be_blk, ed), lambda i: (i, 0)),
        out_shape=jax.ShapeDtypeStruct((ep, ed), jnp.float32),
    )(e, w, b.reshape(1, ed))


def _sc_edge_body(n_chunks, src_ref, dst_ref, psrc_ref, pdst_ref, r_ref,
                  out_ref, si_v, di_v, gs_v, gd_v, r_v, sem_s, sem_d):
    wid = lax.axis_index("s") * 2 + lax.axis_index("c")
    rows_per_chunk = CHUNK // IDX_W
    for c in range(n_chunks):
        base = wid * (n_chunks * CHUNK) + c * CHUNK
        irow = wid * (n_chunks * rows_per_chunk) + c * rows_per_chunk
        pltpu.sync_copy(src_ref.at[pl.ds(irow, rows_per_chunk)], si_v)
        pltpu.sync_copy(dst_ref.at[pl.ds(irow, rows_per_chunk)], di_v)
        pltpu.sync_copy(r_ref.at[pl.ds(base, CHUNK)], r_v)
        handles = []
        for j in range(rows_per_chunk):
            handles.append(pltpu.async_copy(
                psrc_ref.at[si_v.at[j]], gs_v.at[pl.ds(j * IDX_W, IDX_W)],
                sem_s))
            handles.append(pltpu.async_copy(
                pdst_ref.at[di_v.at[j]], gd_v.at[pl.ds(j * IDX_W, IDX_W)],
                sem_d))
        for h in handles:
            h.wait()

        def combine(j, _):
            r_v[j] = jnp.maximum(gs_v[j] + gd_v[j] + r_v[j], 0.0)
            return _

        lax.fori_loop(0, CHUNK, combine, 0)
        pltpu.sync_copy(r_v, out_ref.at[pl.ds(base, CHUNK)])


def _sc_edge(src2d, dst2d, psrc, pdst, r):
    ep, ed = r.shape
    n_chunks = ep // (NW * CHUNK)
    mesh = plsc.VectorSubcoreMesh(core_axis_name="c", subcore_axis_name="s")
    rows_per_chunk = CHUNK // IDX_W
    fn = pl.kernel(
        functools.partial(_sc_edge_body, n_chunks),
        out_type=jax.ShapeDtypeStruct((ep, ed), jnp.float32),
        mesh=mesh,
        compiler_params=pltpu.CompilerParams(use_tc_tiling_on_sc=False),
        scratch_types=[
            pltpu.VMEM((rows_per_chunk, IDX_W), jnp.int32),
            pltpu.VMEM((rows_per_chunk, IDX_W), jnp.int32),
            pltpu.VMEM((CHUNK, ed), jnp.float32),
            pltpu.VMEM((CHUNK, ed), jnp.float32),
            pltpu.VMEM((CHUNK, ed), jnp.float32),
            pltpu.SemaphoreType.DMA,
            pltpu.SemaphoreType.DMA,
        ],
    )
    return fn(src2d, dst2d, psrc, pdst, r)


def kernel(s, edge_index, edge_attr, batch, ln_gamma, ln_beta, We, be,
           Wm, bm, Wn, bn):
    n, d = s.shape
    e_cnt, ed = edge_attr.shape
    L = We.shape[0]

    e_pad = ((e_cnt + NW * CHUNK - 1) // (NW * CHUNK)) * (NW * CHUNK)
    pad = e_pad - e_cnt
    src2d = jnp.concatenate(
        [edge_index[0], jnp.zeros((pad,), jnp.int32)]).reshape(-1, IDX_W)
    dst2d = jnp.concatenate(
        [edge_index[1], jnp.zeros((pad,), jnp.int32)]).reshape(-1, IDX_W)
    e = jnp.concatenate([edge_attr, jnp.zeros((pad, ed), jnp.float32)])

    w2 = jnp.concatenate([We[:, :d, :], We[:, d:2 * d, :]], axis=2)
    wee = We[:, 2 * d:, :]

    batch_r = batch.reshape(n, 1)
    batch_c = batch.reshape(1, n)
    for i in range(L):
        s, psrc, pdst = _ln_layer(s, batch_r, batch_c, ln_gamma[i],
                                  ln_beta[i], w2[i])
        r = _edge_mm(e, wee[i], be[i])
        e = _sc_edge(src2d, dst2d, psrc, pdst, r)

    return (s, e[:e_cnt])


# speedup vs baseline: 6.7527x; 2.3266x over previous
import functools

import jax
import jax.numpy as jnp
from jax import lax
from jax.experimental import pallas as pl
from jax.experimental.pallas import tpu as pltpu
from jax.experimental.pallas import tpu_sc as plsc

G = 64
NW = 32
CHUNK = 1024
IDX_W = 128


def _ln_layer_body(D, eps, s_ref, br_ref, bc_ref, gam_ref, bet_ref, w2_ref,
                   s_out_ref, psrc_ref, pdst_ref):
    n = s_ref.shape[0]
    iota_row = lax.broadcasted_iota(jnp.int32, (1, G), 1)
    iota_col = lax.broadcasted_iota(jnp.int32, (G, 1), 0)
    onehot = (br_ref[...] == iota_row).astype(jnp.float32)
    onehot_t = (iota_col == bc_ref[...]).astype(jnp.float32)
    ones_n = jnp.ones((n, 1), jnp.float32)
    cnt_g = lax.dot_general(onehot_t, ones_n, (((1,), (0,)), ((), ())))
    cnt_d = jnp.maximum(cnt_g * jnp.float32(D), 1.0)
    s = s_ref[...]
    t = jnp.sum(s, axis=1, keepdims=True)
    mean_g = lax.dot_general(onehot_t, t, (((1,), (0,)), ((), ()))) / cnt_d
    mean_n = lax.dot_general(onehot, mean_g, (((1,), (0,)), ((), ())))
    diff = s - mean_n
    sq = jnp.sum(diff * diff, axis=1, keepdims=True)
    var_g = lax.dot_general(onehot_t, sq, (((1,), (0,)), ((), ()))) / cnt_d
    var_n = lax.dot_general(onehot, var_g, (((1,), (0,)), ((), ())))
    s = diff / jnp.sqrt(var_n + eps) * gam_ref[...] + bet_ref[...]
    s_out_ref[...] = s
    p = lax.dot_general(s, w2_ref[...], (((1,), (0,)), ((), ())),
                        preferred_element_type=jnp.float32)
    psrc_ref[...] = p[:, :16]
    pdst_ref[...] = p[:, 16:]


def _ln_layer(s, batch_r, batch_c, gamma, beta, w2_i):
    n, d = s.shape
    body = functools.partial(_ln_layer_body, d, 1e-5)
    return pl.pallas_call(
        body,
        out_shape=[
            jax.ShapeDtypeStruct((n, d), jnp.float32),
            jax.ShapeDtypeStruct((n, 16), jnp.float32),
            jax.ShapeDtypeStruct((n, 16), jnp.float32),
        ],
    )(s, batch_r, batch_c, gamma.reshape(1, d), beta.reshape(1, d), w2_i)


def _edge_mm_body(e_ref, w_ref, b_ref, o_ref):
    o_ref[...] = lax.dot_general(
        e_ref[...], w_ref[...], (((1,), (0,)), ((), ())),
        preferred_element_type=jnp.float32) + b_ref[...]


def _edge_mm(e2d, w8, b8):
    e8 = e2d.shape[0]
    blk = 4096
    return pl.pallas_call(
        _edge_mm_body,
        grid=(e8 // blk,),
        in_specs=[
            pl.BlockSpec((blk, 128), lambda i: (i, 0)),
            pl.BlockSpec((128, 128), lambda i: (0, 0)),
            pl.BlockSpec((1, 128), lambda i: (0, 0)),
        ],
        out_specs=pl.BlockSpec((blk, 128), lambda i: (i, 0)),
        out_shape=jax.ShapeDtypeStruct((e8, 128), jnp.float32),
    )(e2d, w8, b8.reshape(1, 128))


def _sc_edge_body(n_chunks, src_ref, dst_ref, psrc_ref, pdst_ref, r_ref,
                  out_ref, si_v, di_v, gs_v, gd_v, r_v, sem_s, sem_d):
    wid = lax.axis_index("s") * 2 + lax.axis_index("c")
    idx_rows = CHUNK // IDX_W
    p_rows = CHUNK // 8
    for c in range(n_chunks):
        irow = wid * (n_chunks * idx_rows) + c * idx_rows
        prow = wid * (n_chunks * p_rows) + c * p_rows
        pltpu.sync_copy(src_ref.at[pl.ds(irow, idx_rows)], si_v)
        pltpu.sync_copy(dst_ref.at[pl.ds(irow, idx_rows)], di_v)
        pltpu.sync_copy(r_ref.at[pl.ds(prow, p_rows)], r_v)
        handles = []
        for j in range(idx_rows):
            handles.append(pltpu.async_copy(
                psrc_ref.at[si_v.at[j]], gs_v.at[pl.ds(j * IDX_W, IDX_W)],
                sem_s))
            handles.append(pltpu.async_copy(
                pdst_ref.at[di_v.at[j]], gd_v.at[pl.ds(j * IDX_W, IDX_W)],
                sem_d))
        for h in handles:
            h.wait()

        def combine(j, _):
            for k in range(8):
                lane = k * 16
                v = gs_v[j * 8 + k] + gd_v[j * 8 + k] + r_v[j, pl.ds(lane, 16)]
                r_v[j, pl.ds(lane, 16)] = jnp.maximum(v, 0.0)
            return _

        lax.fori_loop(0, p_rows, combine, 0)
        pltpu.sync_copy(r_v, out_ref.at[pl.ds(prow, p_rows)])


def _sc_edge(src2d, dst2d, psrc, pdst, r2d):
    e8 = r2d.shape[0]
    ep = e8 * 8
    n_chunks = ep // (NW * CHUNK)
    mesh = plsc.VectorSubcoreMesh(core_axis_name="c", subcore_axis_name="s")
    fn = pl.kernel(
        functools.partial(_sc_edge_body, n_chunks),
        out_type=jax.ShapeDtypeStruct((e8, 128), jnp.float32),
        mesh=mesh,
        compiler_params=pltpu.CompilerParams(use_tc_tiling_on_sc=False),
        scratch_types=[
            pltpu.VMEM((CHUNK // IDX_W, IDX_W), jnp.int32),
            pltpu.VMEM((CHUNK // IDX_W, IDX_W), jnp.int32),
            pltpu.VMEM((CHUNK, 16), jnp.float32),
            pltpu.VMEM((CHUNK, 16), jnp.float32),
            pltpu.VMEM((CHUNK // 8, 128), jnp.float32),
            pltpu.SemaphoreType.DMA,
            pltpu.SemaphoreType.DMA,
        ],
    )
    return fn(src2d, dst2d, psrc, pdst, r2d)


def kernel(s, edge_index, edge_attr, batch, ln_gamma, ln_beta, We, be,
           Wm, bm, Wn, bn):
    n, d = s.shape
    e_cnt, ed = edge_attr.shape
    L = We.shape[0]

    e_pad = ((e_cnt + NW * CHUNK - 1) // (NW * CHUNK)) * (NW * CHUNK)
    pad = e_pad - e_cnt
    src2d = jnp.concatenate(
        [edge_index[0], jnp.zeros((pad,), jnp.int32)]).reshape(-1, IDX_W)
    dst2d = jnp.concatenate(
        [edge_index[1], jnp.zeros((pad,), jnp.int32)]).reshape(-1, IDX_W)
    e2d = jnp.concatenate(
        [edge_attr, jnp.zeros((pad, ed), jnp.float32)]).reshape(-1, 8 * ed)

    w2 = jnp.concatenate([We[:, :d, :], We[:, d:2 * d, :]], axis=2)
    eye8 = jnp.eye(8, dtype=jnp.float32)
    w8 = jnp.stack([jnp.kron(eye8, We[i, 2 * d:, :]) for i in range(L)])
    b8 = jnp.tile(be, (1, 8))

    batch_r = batch.reshape(n, 1)
    batch_c = batch.reshape(1, n)
    for i in range(L):
        s, psrc, pdst = _ln_layer(s, batch_r, batch_c, ln_gamma[i],
                                  ln_beta[i], w2[i])
        r2d = _edge_mm(e2d, w8[i], b8[i])
        e2d = _sc_edge(src2d, dst2d, psrc, pdst, r2d)

    return (s, e2d.reshape(-1, ed)[:e_cnt])


# speedup vs baseline: 8.2821x; 1.2265x over previous
import functools

import jax
import jax.numpy as jnp
from jax import lax
from jax.experimental import pallas as pl
from jax.experimental.pallas import tpu as pltpu
from jax.experimental.pallas import tpu_sc as plsc

G = 64
NW = 32
CHUNK = 1024
IDX_W = 128


def _ln_layer_body(D, eps, s_ref, br_ref, bc_ref, gam_ref, bet_ref, w2_ref,
                   s_out_ref, psrc_ref, pdst_ref):
    n = s_ref.shape[0]
    iota_row = lax.broadcasted_iota(jnp.int32, (1, G), 1)
    iota_col = lax.broadcasted_iota(jnp.int32, (G, 1), 0)
    onehot = (br_ref[...] == iota_row).astype(jnp.float32)
    onehot_t = (iota_col == bc_ref[...]).astype(jnp.float32)
    ones_n = jnp.ones((n, 1), jnp.float32)
    cnt_g = lax.dot_general(onehot_t, ones_n, (((1,), (0,)), ((), ())))
    cnt_d = jnp.maximum(cnt_g * jnp.float32(D), 1.0)
    s = s_ref[...]
    t = jnp.sum(s, axis=1, keepdims=True)
    mean_g = lax.dot_general(onehot_t, t, (((1,), (0,)), ((), ()))) / cnt_d
    mean_n = lax.dot_general(onehot, mean_g, (((1,), (0,)), ((), ())))
    diff = s - mean_n
    sq = jnp.sum(diff * diff, axis=1, keepdims=True)
    var_g = lax.dot_general(onehot_t, sq, (((1,), (0,)), ((), ()))) / cnt_d
    var_n = lax.dot_general(onehot, var_g, (((1,), (0,)), ((), ())))
    s = diff / jnp.sqrt(var_n + eps) * gam_ref[...] + bet_ref[...]
    s_out_ref[...] = s
    p = lax.dot_general(s, w2_ref[...], (((1,), (0,)), ((), ())),
                        preferred_element_type=jnp.float32)
    psrc_ref[...] = p[:, :16]
    pdst_ref[...] = p[:, 16:]


def _ln_layer(s, batch_r, batch_c, gamma, beta, w2_i):
    n, d = s.shape
    body = functools.partial(_ln_layer_body, d, 1e-5)
    return pl.pallas_call(
        body,
        out_shape=[
            jax.ShapeDtypeStruct((n, d), jnp.float32),
            jax.ShapeDtypeStruct((n, 16), jnp.float32),
            jax.ShapeDtypeStruct((n, 16), jnp.float32),
        ],
    )(s, batch_r, batch_c, gamma.reshape(1, d), beta.reshape(1, d), w2_i)


def _edge_mm_body(e_ref, w_ref, b_ref, o_ref):
    o_ref[...] = lax.dot_general(
        e_ref[...], w_ref[...], (((1,), (0,)), ((), ())),
        preferred_element_type=jnp.float32) + b_ref[...]


def _edge_mm(e2d, w8, b8):
    e8 = e2d.shape[0]
    blk = 4096
    return pl.pallas_call(
        _edge_mm_body,
        grid=(e8 // blk,),
        in_specs=[
            pl.BlockSpec((blk, 128), lambda i: (i, 0)),
            pl.BlockSpec((128, 128), lambda i: (0, 0)),
            pl.BlockSpec((1, 128), lambda i: (0, 0)),
        ],
        out_specs=pl.BlockSpec((blk, 128), lambda i: (i, 0)),
        out_shape=jax.ShapeDtypeStruct((e8, 128), jnp.float32),
    )(e2d, w8, b8.reshape(1, 128))


def _sc_edge_body(n_chunks, src_ref, dst_ref, psrc_ref, pdst_ref, r_ref,
                  out_ref,
                  si_v, di_v, gs_v, gd_v, r_v, sem_g, sem_r, sem_o):
    wid = lax.axis_index("s") * 2 + lax.axis_index("c")
    idx_rows = CHUNK // IDX_W
    p_rows = CHUNK // 8

    load_handles = {}

    def stage_load(c):
        b = c % 2
        irow = wid * (n_chunks * idx_rows) + c * idx_rows
        prow = wid * (n_chunks * p_rows) + c * p_rows
        pltpu.sync_copy(src_ref.at[pl.ds(irow, idx_rows)], si_v[b])
        pltpu.sync_copy(dst_ref.at[pl.ds(irow, idx_rows)], di_v[b])
        hs = [pltpu.async_copy(r_ref.at[pl.ds(prow, p_rows)], r_v[b],
                               sem_r[b])]
        for j in range(idx_rows):
            hs.append(pltpu.async_copy(
                psrc_ref.at[si_v[b].at[j]],
                gs_v[b].at[pl.ds(j * IDX_W, IDX_W)], sem_g[b]))
            hs.append(pltpu.async_copy(
                pdst_ref.at[di_v[b].at[j]],
                gd_v[b].at[pl.ds(j * IDX_W, IDX_W)], sem_g[b]))
        load_handles[c] = hs

    store_handles = {}
    stage_load(0)
    for c in range(n_chunks):
        b = c % 2
        if c + 1 < n_chunks:
            if c - 1 >= 0:
                store_handles.pop(c - 1).wait()
            stage_load(c + 1)
        for h in load_handles.pop(c):
            h.wait()

        gsb, gdb, rvb = gs_v[b], gd_v[b], r_v[b]

        def combine(j, _):
            for k in range(8):
                lane = k * 16
                v = gsb[j * 8 + k] + gdb[j * 8 + k] + rvb[j, pl.ds(lane, 16)]
                rvb[j, pl.ds(lane, 16)] = jnp.maximum(v, 0.0)
            return _

        lax.fori_loop(0, p_rows, combine, 0)
        prow = wid * (n_chunks * p_rows) + c * p_rows
        store_handles[c] = pltpu.async_copy(
            r_v[b], out_ref.at[pl.ds(prow, p_rows)], sem_o[b])
    for c in sorted(store_handles):
        store_handles.pop(c).wait()


def _sc_edge(src2d, dst2d, psrc, pdst, r2d):
    e8 = r2d.shape[0]
    ep = e8 * 8
    n_chunks = ep // (NW * CHUNK)
    mesh = plsc.VectorSubcoreMesh(core_axis_name="c", subcore_axis_name="s")
    fn = pl.kernel(
        functools.partial(_sc_edge_body, n_chunks),
        out_type=jax.ShapeDtypeStruct((e8, 128), jnp.float32),
        mesh=mesh,
        compiler_params=pltpu.CompilerParams(use_tc_tiling_on_sc=False),
        scratch_types=[
            [pltpu.VMEM((CHUNK // IDX_W, IDX_W), jnp.int32) for _ in range(2)],
            [pltpu.VMEM((CHUNK // IDX_W, IDX_W), jnp.int32) for _ in range(2)],
            [pltpu.VMEM((CHUNK, 16), jnp.float32) for _ in range(2)],
            [pltpu.VMEM((CHUNK, 16), jnp.float32) for _ in range(2)],
            [pltpu.VMEM((CHUNK // 8, 128), jnp.float32) for _ in range(2)],
            [pltpu.SemaphoreType.DMA for _ in range(2)],
            [pltpu.SemaphoreType.DMA for _ in range(2)],
            [pltpu.SemaphoreType.DMA for _ in range(2)],
        ],
    )
    return fn(src2d, dst2d, psrc, pdst, r2d)


def kernel(s, edge_index, edge_attr, batch, ln_gamma, ln_beta, We, be,
           Wm, bm, Wn, bn):
    n, d = s.shape
    e_cnt, ed = edge_attr.shape
    L = We.shape[0]

    e_pad = ((e_cnt + NW * CHUNK - 1) // (NW * CHUNK)) * (NW * CHUNK)
    pad = e_pad - e_cnt
    src2d = jnp.concatenate(
        [edge_index[0], jnp.zeros((pad,), jnp.int32)]).reshape(-1, IDX_W)
    dst2d = jnp.concatenate(
        [edge_index[1], jnp.zeros((pad,), jnp.int32)]).reshape(-1, IDX_W)
    e2d = jnp.concatenate(
        [edge_attr.reshape(-1, 8 * ed),
         jnp.zeros((pad // 8, 8 * ed), jnp.float32)])

    w2 = jnp.concatenate([We[:, :d, :], We[:, d:2 * d, :]], axis=2)
    eye8 = jnp.eye(8, dtype=jnp.float32)
    w8 = jnp.stack([jnp.kron(eye8, We[i, 2 * d:, :]) for i in range(L)])
    b8 = jnp.tile(be, (1, 8))

    batch_r = batch.reshape(n, 1)
    batch_c = batch.reshape(1, n)
    tables = []
    for i in range(L):
        s, psrc, pdst = _ln_layer(s, batch_r, batch_c, ln_gamma[i],
                                  ln_beta[i], w2[i])
        tables.append((psrc, pdst))
    for i in range(L):
        r2d = _edge_mm(e2d, w8[i], b8[i])
        e2d = _sc_edge(src2d, dst2d, tables[i][0], tables[i][1], r2d)

    return (s, e2d.reshape(-1, ed)[:e_cnt])


# speedup vs baseline: 10.8396x; 1.3088x over previous
import functools

import jax
import jax.numpy as jnp
from jax import lax
from jax.experimental import pallas as pl
from jax.experimental.pallas import tpu as pltpu
from jax.experimental.pallas import tpu_sc as plsc

G = 64
NW = 32
CHUNK = 1000
IDX_W = 128


def _ln_layer_body(D, eps, s_ref, br_ref, bc_ref, gam_ref, bet_ref, w2_ref,
                   s_out_ref, psrc_ref, pdst_ref):
    n = s_ref.shape[0]
    iota_row = lax.broadcasted_iota(jnp.int32, (1, G), 1)
    iota_col = lax.broadcasted_iota(jnp.int32, (G, 1), 0)
    onehot = (br_ref[...] == iota_row).astype(jnp.float32)
    onehot_t = (iota_col == bc_ref[...]).astype(jnp.float32)
    ones_n = jnp.ones((n, 1), jnp.float32)
    cnt_g = lax.dot_general(onehot_t, ones_n, (((1,), (0,)), ((), ())))
    cnt_d = jnp.maximum(cnt_g * jnp.float32(D), 1.0)
    s = s_ref[...]
    t = jnp.sum(s, axis=1, keepdims=True)
    mean_g = lax.dot_general(onehot_t, t, (((1,), (0,)), ((), ()))) / cnt_d
    mean_n = lax.dot_general(onehot, mean_g, (((1,), (0,)), ((), ())))
    diff = s - mean_n
    sq = jnp.sum(diff * diff, axis=1, keepdims=True)
    var_g = lax.dot_general(onehot_t, sq, (((1,), (0,)), ((), ()))) / cnt_d
    var_n = lax.dot_general(onehot, var_g, (((1,), (0,)), ((), ())))
    s = diff / jnp.sqrt(var_n + eps) * gam_ref[...] + bet_ref[...]
    s_out_ref[...] = s
    p = lax.dot_general(s, w2_ref[...], (((1,), (0,)), ((), ())),
                        preferred_element_type=jnp.float32)
    psrc_ref[...] = p[:, :16]
    pdst_ref[...] = p[:, 16:]


def _ln_layer(s, batch_r, batch_c, gamma, beta, w2_i):
    n, d = s.shape
    body = functools.partial(_ln_layer_body, d, 1e-5)
    return pl.pallas_call(
        body,
        out_shape=[
            jax.ShapeDtypeStruct((n, d), jnp.float32),
            jax.ShapeDtypeStruct((n, 16), jnp.float32),
            jax.ShapeDtypeStruct((n, 16), jnp.float32),
        ],
    )(s, batch_r, batch_c, gamma.reshape(1, d), beta.reshape(1, d), w2_i)


def _edge_mm_body(e_ref, w_ref, b_ref, o_ref):
    o_ref[...] = lax.dot_general(
        e_ref[...], w_ref[...], (((1,), (0,)), ((), ())),
        preferred_element_type=jnp.float32) + b_ref[...]


def _edge_mm(e2d, w8, b8):
    e8 = e2d.shape[0]
    blk = 4000
    return pl.pallas_call(
        _edge_mm_body,
        grid=(e8 // blk,),
        in_specs=[
            pl.BlockSpec((blk, 128), lambda i: (i, 0)),
            pl.BlockSpec((128, 128), lambda i: (0, 0)),
            pl.BlockSpec((1, 128), lambda i: (0, 0)),
        ],
        out_specs=pl.BlockSpec((blk, 128), lambda i: (i, 0)),
        out_shape=jax.ShapeDtypeStruct((e8, 128), jnp.float32),
    )(e2d, w8, b8.reshape(1, 128))


def _edge_mm_first_body(e_ref, w_ref, b_ref, o_ref):
    cols = [e_ref[k::8, :] for k in range(8)]
    packed = jnp.concatenate(cols, axis=1)
    o_ref[...] = lax.dot_general(
        packed, w_ref[...], (((1,), (0,)), ((), ())),
        preferred_element_type=jnp.float32) + b_ref[...]


def _edge_mm_first(e, w8, b8):
    e_cnt, ed = e.shape
    blk = 8000
    return pl.pallas_call(
        _edge_mm_first_body,
        grid=(e_cnt // blk,),
        in_specs=[
            pl.BlockSpec((blk, ed), lambda i: (i, 0)),
            pl.BlockSpec((128, 128), lambda i: (0, 0)),
            pl.BlockSpec((1, 128), lambda i: (0, 0)),
        ],
        out_specs=pl.BlockSpec((blk // 8, 128), lambda i: (i, 0)),
        out_shape=jax.ShapeDtypeStruct((e_cnt // 8, 128), jnp.float32),
    )(e, w8, b8.reshape(1, 128))


def _unpack_body(e2_ref, o_ref):
    for k in range(8):
        o_ref[k::8, :] = e2_ref[:, k * 16:(k + 1) * 16]


def _unpack(e2d, ed):
    e8 = e2d.shape[0]
    blk = 1000
    return pl.pallas_call(
        _unpack_body,
        grid=(e8 // blk,),
        in_specs=[pl.BlockSpec((blk, 128), lambda i: (i, 0))],
        out_specs=pl.BlockSpec((blk * 8, ed), lambda i: (i, 0)),
        out_shape=jax.ShapeDtypeStruct((e8 * 8, ed), jnp.float32),
    )(e2d)


def _sc_edge_body(n_chunks, src_ref, dst_ref, psrc_ref, pdst_ref, r_ref,
                  out_ref,
                  si_v, di_v, gs_v, gd_v, r_v, sem_g, sem_r, sem_o):
    wid = lax.axis_index("s") * 2 + lax.axis_index("c")
    p_rows = CHUNK // 8
    n_full = CHUNK // IDX_W
    tail = CHUNK - n_full * IDX_W

    load_handles = {}

    def stage_load(c):
        b = c % 2
        ibase = wid * (n_chunks * CHUNK) + c * CHUNK
        prow = wid * (n_chunks * p_rows) + c * p_rows
        pltpu.sync_copy(src_ref.at[pl.ds(ibase, CHUNK)], si_v[b])
        pltpu.sync_copy(dst_ref.at[pl.ds(ibase, CHUNK)], di_v[b])
        hs = [pltpu.async_copy(r_ref.at[pl.ds(prow, p_rows)], r_v[b],
                               sem_r[b])]
        spans = [(j * IDX_W, IDX_W) for j in range(n_full)]
        if tail:
            spans.append((n_full * IDX_W, tail))
        for off, w in spans:
            hs.append(pltpu.async_copy(
                psrc_ref.at[si_v[b].at[pl.ds(off, w)]],
                gs_v[b].at[pl.ds(off, w)], sem_g[b]))
            hs.append(pltpu.async_copy(
                pdst_ref.at[di_v[b].at[pl.ds(off, w)]],
                gd_v[b].at[pl.ds(off, w)], sem_g[b]))
        load_handles[c] = hs

    store_handles = {}
    stage_load(0)
    for c in range(n_chunks):
        b = c % 2
        if c + 1 < n_chunks:
            if c - 1 >= 0:
                store_handles.pop(c - 1).wait()
            stage_load(c + 1)
        for h in load_handles.pop(c):
            h.wait()

        gsb, gdb, rvb = gs_v[b], gd_v[b], r_v[b]

        def combine(j, _):
            for k in range(8):
                lane = k * 16
                v = gsb[j * 8 + k] + gdb[j * 8 + k] + rvb[j, pl.ds(lane, 16)]
                rvb[j, pl.ds(lane, 16)] = jnp.maximum(v, 0.0)
            return _

        lax.fori_loop(0, p_rows, combine, 0)
        prow = wid * (n_chunks * p_rows) + c * p_rows
        store_handles[c] = pltpu.async_copy(
            r_v[b], out_ref.at[pl.ds(prow, p_rows)], sem_o[b])
    for c in sorted(store_handles):
        store_handles.pop(c).wait()


def _sc_edge(src, dst, psrc, pdst, r2d):
    e8 = r2d.shape[0]
    n_chunks = (e8 * 8) // (NW * CHUNK)
    mesh = plsc.VectorSubcoreMesh(core_axis_name="c", subcore_axis_name="s")
    fn = pl.kernel(
        functools.partial(_sc_edge_body, n_chunks),
        out_type=jax.ShapeDtypeStruct((e8, 128), jnp.float32),
        mesh=mesh,
        compiler_params=pltpu.CompilerParams(use_tc_tiling_on_sc=False),
        scratch_types=[
            [pltpu.VMEM((CHUNK,), jnp.int32) for _ in range(2)],
            [pltpu.VMEM((CHUNK,), jnp.int32) for _ in range(2)],
            [pltpu.VMEM((CHUNK, 16), jnp.float32) for _ in range(2)],
            [pltpu.VMEM((CHUNK, 16), jnp.float32) for _ in range(2)],
            [pltpu.VMEM((CHUNK // 8, 128), jnp.float32) for _ in range(2)],
            [pltpu.SemaphoreType.DMA for _ in range(2)],
            [pltpu.SemaphoreType.DMA for _ in range(2)],
            [pltpu.SemaphoreType.DMA for _ in range(2)],
        ],
    )
    return fn(src, dst, psrc, pdst, r2d)


def kernel(s, edge_index, edge_attr, batch, ln_gamma, ln_beta, We, be,
           Wm, bm, Wn, bn):
    n, d = s.shape
    e_cnt, ed = edge_attr.shape
    L = We.shape[0]

    src = edge_index[0]
    dst = edge_index[1]

    w2 = jnp.concatenate([We[:, :d, :], We[:, d:2 * d, :]], axis=2)
    eye8 = jnp.eye(8, dtype=jnp.float32)
    w8 = jnp.stack([jnp.kron(eye8, We[i, 2 * d:, :]) for i in range(L)])
    b8 = jnp.tile(be, (1, 8))

    batch_r = batch.reshape(n, 1)
    batch_c = batch.reshape(1, n)
    tables = []
    for i in range(L):
        s, psrc, pdst = _ln_layer(s, batch_r, batch_c, ln_gamma[i],
                                  ln_beta[i], w2[i])
        tables.append((psrc, pdst))

    e2d = None
    for i in range(L):
        if i == 0:
            r2d = _edge_mm_first(edge_attr, w8[0], b8[0])
        else:
            r2d = _edge_mm(e2d, w8[i], b8[i])
        e2d = _sc_edge(src, dst, tables[i][0], tables[i][1], r2d)

    return (s, _unpack(e2d, ed))


# speedup vs baseline: 11.5364x; 1.0643x over previous
import functools

import jax
import jax.numpy as jnp
from jax import lax
from jax.experimental import pallas as pl
from jax.experimental.pallas import tpu as pltpu
from jax.experimental.pallas import tpu_sc as plsc

G = 64
NW = 32
CHUNK = 1000
IDX_W = 128


def _ln_layer_body(D, eps, s_ref, br_ref, bc_ref, gam_ref, bet_ref, w2_ref,
                   s_out_ref, psrc_ref, pdst_ref):
    n = s_ref.shape[0]
    iota_row = lax.broadcasted_iota(jnp.int32, (1, G), 1)
    iota_col = lax.broadcasted_iota(jnp.int32, (G, 1), 0)
    onehot = (br_ref[...] == iota_row).astype(jnp.float32)
    onehot_t = (iota_col == bc_ref[...]).astype(jnp.float32)
    ones_n = jnp.ones((n, 1), jnp.float32)
    cnt_g = lax.dot_general(onehot_t, ones_n, (((1,), (0,)), ((), ())))
    cnt_d = jnp.maximum(cnt_g * jnp.float32(D), 1.0)
    s = s_ref[...]
    t = jnp.sum(s, axis=1, keepdims=True)
    mean_g = lax.dot_general(onehot_t, t, (((1,), (0,)), ((), ()))) / cnt_d
    mean_n = lax.dot_general(onehot, mean_g, (((1,), (0,)), ((), ())))
    diff = s - mean_n
    sq = jnp.sum(diff * diff, axis=1, keepdims=True)
    var_g = lax.dot_general(onehot_t, sq, (((1,), (0,)), ((), ()))) / cnt_d
    var_n = lax.dot_general(onehot, var_g, (((1,), (0,)), ((), ())))
    s = diff / jnp.sqrt(var_n + eps) * gam_ref[...] + bet_ref[...]
    s_out_ref[...] = s
    p = lax.dot_general(s, w2_ref[...], (((1,), (0,)), ((), ())),
                        preferred_element_type=jnp.float32)
    psrc_ref[...] = p[:, :16]
    pdst_ref[...] = p[:, 16:]


def _ln_layer(s, batch_r, batch_c, gamma, beta, w2_i):
    n, d = s.shape
    body = functools.partial(_ln_layer_body, d, 1e-5)
    return pl.pallas_call(
        body,
        out_shape=[
            jax.ShapeDtypeStruct((n, d), jnp.float32),
            jax.ShapeDtypeStruct((n, 16), jnp.float32),
            jax.ShapeDtypeStruct((n, 16), jnp.float32),
        ],
    )(s, batch_r, batch_c, gamma.reshape(1, d), beta.reshape(1, d), w2_i)


def _edge_mm_body(e_ref, w_ref, b_ref, o_ref):
    o_ref[...] = lax.dot_general(
        e_ref[...], w_ref[...], (((1,), (0,)), ((), ())),
        preferred_element_type=jnp.float32) + b_ref[...]


def _edge_mm(e2d, w8, b8):
    e8 = e2d.shape[0]
    blk = 4000
    return pl.pallas_call(
        _edge_mm_body,
        grid=(e8 // blk,),
        in_specs=[
            pl.BlockSpec((blk, 128), lambda i: (i, 0)),
            pl.BlockSpec((128, 128), lambda i: (0, 0)),
            pl.BlockSpec((1, 128), lambda i: (0, 0)),
        ],
        out_specs=pl.BlockSpec((blk, 128), lambda i: (i, 0)),
        out_shape=jax.ShapeDtypeStruct((e8, 128), jnp.float32),
    )(e2d, w8, b8.reshape(1, 128))


def _edge_mm_first_body(e_ref, w_ref, b_ref, o_ref):
    cols = [e_ref[k::8, :] for k in range(8)]
    packed = jnp.concatenate(cols, axis=1)
    o_ref[...] = lax.dot_general(
        packed, w_ref[...], (((1,), (0,)), ((), ())),
        preferred_element_type=jnp.float32) + b_ref[...]


def _edge_mm_first(e, w8, b8):
    e_cnt, ed = e.shape
    blk = 8000
    return pl.pallas_call(
        _edge_mm_first_body,
        grid=(e_cnt // blk,),
        in_specs=[
            pl.BlockSpec((blk, ed), lambda i: (i, 0)),
            pl.BlockSpec((128, 128), lambda i: (0, 0)),
            pl.BlockSpec((1, 128), lambda i: (0, 0)),
        ],
        out_specs=pl.BlockSpec((blk // 8, 128), lambda i: (i, 0)),
        out_shape=jax.ShapeDtypeStruct((e_cnt // 8, 128), jnp.float32),
    )(e, w8, b8.reshape(1, 128))


def _unpack_body(e2_ref, o_ref):
    for k in range(8):
        o_ref[k::8, :] = e2_ref[:, k * 16:(k + 1) * 16]


def _unpack(e2d, ed):
    e8 = e2d.shape[0]
    blk = 1000
    return pl.pallas_call(
        _unpack_body,
        grid=(e8 // blk,),
        in_specs=[pl.BlockSpec((blk, 128), lambda i: (i, 0))],
        out_specs=pl.BlockSpec((blk * 8, ed), lambda i: (i, 0)),
        out_shape=jax.ShapeDtypeStruct((e8 * 8, ed), jnp.float32),
    )(e2d)


def _sc_edge_body(n_chunks, src_ref, dst_ref, psrc_ref, pdst_ref, r_ref,
                  out_ref,
                  si_v, di_v, gs_v, gd_v, r_v, sem_g, sem_r, sem_o):
    wid = lax.axis_index("s") * 2 + lax.axis_index("c")
    p_rows = CHUNK // 8
    n_full = CHUNK // IDX_W
    tail = CHUNK - n_full * IDX_W

    load_handles = {}

    def stage_load(c):
        b = c % 2
        ibase = wid * (n_chunks * CHUNK) + c * CHUNK
        prow = wid * (n_chunks * p_rows) + c * p_rows
        pltpu.sync_copy(src_ref.at[pl.ds(ibase, CHUNK)], si_v[b])
        pltpu.sync_copy(dst_ref.at[pl.ds(ibase, CHUNK)], di_v[b])
        hs = [pltpu.async_copy(r_ref.at[pl.ds(prow, p_rows)], r_v[b],
                               sem_r[b])]
        spans = [(j * IDX_W, IDX_W) for j in range(n_full)]
        if tail:
            spans.append((n_full * IDX_W, tail))
        for off, w in spans:
            hs.append(pltpu.async_copy(
                psrc_ref.at[si_v[b].at[pl.ds(off, w)]],
                gs_v[b].at[pl.ds(off, w)], sem_g[b]))
            hs.append(pltpu.async_copy(
                pdst_ref.at[di_v[b].at[pl.ds(off, w)]],
                gd_v[b].at[pl.ds(off, w)], sem_g[b]))
        load_handles[c] = hs

    store_handles = {}
    stage_load(0)
    for c in range(n_chunks):
        b = c % 2
        if c + 1 < n_chunks:
            if c - 1 >= 0:
                store_handles.pop(c - 1).wait()
            stage_load(c + 1)
        for h in load_handles.pop(c):
            h.wait()

        gsb, gdb, rvb = gs_v[b], gd_v[b], r_v[b]

        def combine(j, _):
            for k in range(8):
                lane = k * 16
                v = gsb[j * 8 + k] + gdb[j * 8 + k] + rvb[j, pl.ds(lane, 16)]
                rvb[j, pl.ds(lane, 16)] = jnp.maximum(v, 0.0)
            return _

        lax.fori_loop(0, p_rows, combine, 0)
        prow = wid * (n_chunks * p_rows) + c * p_rows
        store_handles[c] = pltpu.async_copy(
            r_v[b], out_ref.at[pl.ds(prow, p_rows)], sem_o[b])
    for c in sorted(store_handles):
        store_handles.pop(c).wait()


def _sc_edge(src, dst, psrc, pdst, r2d):
    e8 = r2d.shape[0]
    n_chunks = (e8 * 8) // (NW * CHUNK)
    mesh = plsc.VectorSubcoreMesh(core_axis_name="c", subcore_axis_name="s")
    fn = pl.kernel(
        functools.partial(_sc_edge_body, n_chunks),
        out_type=jax.ShapeDtypeStruct((e8, 128), jnp.float32),
        mesh=mesh,
        compiler_params=pltpu.CompilerParams(use_tc_tiling_on_sc=False),
        scratch_types=[
            [pltpu.VMEM((CHUNK,), jnp.int32) for _ in range(2)],
            [pltpu.VMEM((CHUNK,), jnp.int32) for _ in range(2)],
            [pltpu.VMEM((CHUNK, 16), jnp.float32) for _ in range(2)],
            [pltpu.VMEM((CHUNK, 16), jnp.float32) for _ in range(2)],
            [pltpu.VMEM((CHUNK // 8, 128), jnp.float32) for _ in range(2)],
            [pltpu.SemaphoreType.DMA for _ in range(2)],
            [pltpu.SemaphoreType.DMA for _ in range(2)],
            [pltpu.SemaphoreType.DMA for _ in range(2)],
        ],
    )
    return fn(src, dst, psrc, pdst, r2d)


def kernel(s, edge_index, edge_attr, batch, ln_gamma, ln_beta, We, be,
           Wm, bm, Wn, bn):
    n, d = s.shape
    e_cnt, ed = edge_attr.shape
    L = We.shape[0]

    src = edge_index[0]
    dst = edge_index[1]

    w2 = jnp.concatenate([We[:, :d, :], We[:, d:2 * d, :]], axis=2)
    eye8 = jnp.eye(8, dtype=jnp.float32)
    w8 = jnp.stack([jnp.kron(eye8, We[i, 2 * d:, :]) for i in range(L)])
    b8 = jnp.tile(be, (1, 8))

    batch_r = batch.reshape(n, 1)
    batch_c = batch.reshape(1, n)
    tables = []
    for i in range(L):
        s, psrc, pdst = _ln_layer(s, batch_r, batch_c, ln_gamma[i],
                                  ln_beta[i], w2[i])
        tables.append((psrc, pdst))

    e2d = edge_attr.reshape(e_cnt // 8, 8 * ed)
    for i in range(L):
        r2d = _edge_mm(e2d, w8[i], b8[i])
        e2d = _sc_edge(src, dst, tables[i][0], tables[i][1], r2d)

    return (s, e2d.reshape(e_cnt, ed))


# speedup vs baseline: 12.2346x; 1.0605x over previous
import functools

import jax
import jax.numpy as jnp
from jax import lax
from jax.experimental import pallas as pl
from jax.experimental.pallas import tpu as pltpu
from jax.experimental.pallas import tpu_sc as plsc

G = 64
NW = 32
CHUNK = 1000
IDX_W = 128


def _ln_layer_body(D, eps, s_ref, br_ref, bc_ref, gam_ref, bet_ref, w2_ref,
                   s_out_ref, psrc_ref, pdst_ref):
    n = s_ref.shape[0]
    iota_row = lax.broadcasted_iota(jnp.int32, (1, G), 1)
    iota_col = lax.broadcasted_iota(jnp.int32, (G, 1), 0)
    onehot = (br_ref[...] == iota_row).astype(jnp.float32)
    onehot_t = (iota_col == bc_ref[...]).astype(jnp.float32)
    ones_n = jnp.ones((n, 1), jnp.float32)
    cnt_g = lax.dot_general(onehot_t, ones_n, (((1,), (0,)), ((), ())))
    cnt_d = jnp.maximum(cnt_g * jnp.float32(D), 1.0)
    s = s_ref[...]
    t = jnp.sum(s, axis=1, keepdims=True)
    mean_g = lax.dot_general(onehot_t, t, (((1,), (0,)), ((), ()))) / cnt_d
    mean_n = lax.dot_general(onehot, mean_g, (((1,), (0,)), ((), ())))
    diff = s - mean_n
    sq = jnp.sum(diff * diff, axis=1, keepdims=True)
    var_g = lax.dot_general(onehot_t, sq, (((1,), (0,)), ((), ()))) / cnt_d
    var_n = lax.dot_general(onehot, var_g, (((1,), (0,)), ((), ())))
    s = diff / jnp.sqrt(var_n + eps) * gam_ref[...] + bet_ref[...]
    s_out_ref[...] = s
    p = lax.dot_general(s, w2_ref[...], (((1,), (0,)), ((), ())),
                        preferred_element_type=jnp.float32)
    psrc_ref[...] = p[:, :16]
    pdst_ref[...] = p[:, 16:]


def _ln_layer(s, batch_r, batch_c, gamma, beta, w2_i):
    n, d = s.shape
    body = functools.partial(_ln_layer_body, d, 1e-5)
    return pl.pallas_call(
        body,
        out_shape=[
            jax.ShapeDtypeStruct((n, d), jnp.float32),
            jax.ShapeDtypeStruct((n, 16), jnp.float32),
            jax.ShapeDtypeStruct((n, 16), jnp.float32),
        ],
    )(s, batch_r, batch_c, gamma.reshape(1, d), beta.reshape(1, d), w2_i)


def _edge_mm_body(e_ref, w_ref, b_ref, o_ref):
    o_ref[...] = lax.dot_general(
        e_ref[...], w_ref[...], (((1,), (0,)), ((), ())),
        preferred_element_type=jnp.float32) + b_ref[...]


def _edge_mm(e2d, w8, b8):
    e8 = e2d.shape[0]
    blk = 4000
    return pl.pallas_call(
        _edge_mm_body,
        grid=(e8 // blk,),
        in_specs=[
            pl.BlockSpec((blk, 128), lambda i: (i, 0)),
            pl.BlockSpec((128, 128), lambda i: (0, 0)),
            pl.BlockSpec((1, 128), lambda i: (0, 0)),
        ],
        out_specs=pl.BlockSpec((blk, 128), lambda i: (i, 0)),
        out_shape=jax.ShapeDtypeStruct((e8, 128), jnp.float32),
    )(e2d, w8, b8.reshape(1, 128))


def _edge_mm_first_body(e_ref, w_ref, b_ref, o_ref):
    cols = [e_ref[k::8, :] for k in range(8)]
    packed = jnp.concatenate(cols, axis=1)
    o_ref[...] = lax.dot_general(
        packed, w_ref[...], (((1,), (0,)), ((), ())),
        preferred_element_type=jnp.float32) + b_ref[...]


def _edge_mm_first(e, w8, b8):
    e_cnt, ed = e.shape
    blk = 8000
    return pl.pallas_call(
        _edge_mm_first_body,
        grid=(e_cnt // blk,),
        in_specs=[
            pl.BlockSpec((blk, ed), lambda i: (i, 0)),
            pl.BlockSpec((128, 128), lambda i: (0, 0)),
            pl.BlockSpec((1, 128), lambda i: (0, 0)),
        ],
        out_specs=pl.BlockSpec((blk // 8, 128), lambda i: (i, 0)),
        out_shape=jax.ShapeDtypeStruct((e_cnt // 8, 128), jnp.float32),
    )(e, w8, b8.reshape(1, 128))


def _unpack_body(e2_ref, o_ref):
    for k in range(8):
        o_ref[k::8, :] = e2_ref[:, k * 16:(k + 1) * 16]


def _unpack(e2d, ed):
    e8 = e2d.shape[0]
    blk = 1000
    return pl.pallas_call(
        _unpack_body,
        grid=(e8 // blk,),
        in_specs=[pl.BlockSpec((blk, 128), lambda i: (i, 0))],
        out_specs=pl.BlockSpec((blk * 8, ed), lambda i: (i, 0)),
        out_shape=jax.ShapeDtypeStruct((e8 * 8, ed), jnp.float32),
    )(e2d)


def _sc_edge_body(n_chunks, src_ref, dst_ref, psrc_ref, pdst_ref, r_ref,
                  out_ref,
                  sh_src, sh_dst, si_v, di_v, gs_v, gd_v, r_v,
                  sem_g, sem_r, sem_o):
    wid = lax.axis_index("s") * 2 + lax.axis_index("c")
    p_rows = CHUNK // 8
    n_full = CHUNK // IDX_W
    tail = CHUNK - n_full * IDX_W

    @pl.when(lax.axis_index("s") == 0)
    def _stage():
        pltpu.sync_copy(psrc_ref, sh_src)
        pltpu.sync_copy(pdst_ref, sh_dst)

    plsc.subcore_barrier()

    load_handles = {}

    def stage_load(c):
        b = c % 2
        ibase = wid * (n_chunks * CHUNK) + c * CHUNK
        prow = wid * (n_chunks * p_rows) + c * p_rows
        pltpu.sync_copy(src_ref.at[pl.ds(ibase, CHUNK)], si_v[b])
        pltpu.sync_copy(dst_ref.at[pl.ds(ibase, CHUNK)], di_v[b])
        hs = [pltpu.async_copy(r_ref.at[pl.ds(prow, p_rows)], r_v[b],
                               sem_r[b])]
        spans = [(j * IDX_W, IDX_W) for j in range(n_full)]
        if tail:
            spans.append((n_full * IDX_W, tail))
        for off, w in spans:
            hs.append(pltpu.async_copy(
                sh_src.at[si_v[b].at[pl.ds(off, w)]],
                gs_v[b].at[pl.ds(off, w)], sem_g[b]))
            hs.append(pltpu.async_copy(
                sh_dst.at[di_v[b].at[pl.ds(off, w)]],
                gd_v[b].at[pl.ds(off, w)], sem_g[b]))
        load_handles[c] = hs

    store_handles = {}
    stage_load(0)
    for c in range(n_chunks):
        b = c % 2
        if c + 1 < n_chunks:
            if c - 1 >= 0:
                store_handles.pop(c - 1).wait()
            stage_load(c + 1)
        for h in load_handles.pop(c):
            h.wait()

        gsb, gdb, rvb = gs_v[b], gd_v[b], r_v[b]

        def combine(j, _):
            for k in range(8):
                lane = k * 16
                v = gsb[j * 8 + k] + gdb[j * 8 + k] + rvb[j, pl.ds(lane, 16)]
                rvb[j, pl.ds(lane, 16)] = jnp.maximum(v, 0.0)
            return _

        lax.fori_loop(0, p_rows, combine, 0)
        prow = wid * (n_chunks * p_rows) + c * p_rows
        store_handles[c] = pltpu.async_copy(
            r_v[b], out_ref.at[pl.ds(prow, p_rows)], sem_o[b])
    for c in sorted(store_handles):
        store_handles.pop(c).wait()


def _sc_edge(src, dst, psrc, pdst, r2d):
    e8 = r2d.shape[0]
    n_chunks = (e8 * 8) // (NW * CHUNK)
    mesh = plsc.VectorSubcoreMesh(core_axis_name="c", subcore_axis_name="s")
    fn = pl.kernel(
        functools.partial(_sc_edge_body, n_chunks),
        out_type=jax.ShapeDtypeStruct((e8, 128), jnp.float32),
        mesh=mesh,
        compiler_params=pltpu.CompilerParams(use_tc_tiling_on_sc=False),
        scratch_types=[
            pltpu.VMEM_SHARED(psrc.shape, jnp.float32),
            pltpu.VMEM_SHARED(pdst.shape, jnp.float32),
            [pltpu.VMEM((CHUNK,), jnp.int32) for _ in range(2)],
            [pltpu.VMEM((CHUNK,), jnp.int32) for _ in range(2)],
            [pltpu.VMEM((CHUNK, 16), jnp.float32) for _ in range(2)],
            [pltpu.VMEM((CHUNK, 16), jnp.float32) for _ in range(2)],
            [pltpu.VMEM((CHUNK // 8, 128), jnp.float32) for _ in range(2)],
            [pltpu.SemaphoreType.DMA for _ in range(2)],
            [pltpu.SemaphoreType.DMA for _ in range(2)],
            [pltpu.SemaphoreType.DMA for _ in range(2)],
        ],
    )
    return fn(src, dst, psrc, pdst, r2d)


def kernel(s, edge_index, edge_attr, batch, ln_gamma, ln_beta, We, be,
           Wm, bm, Wn, bn):
    n, d = s.shape
    e_cnt, ed = edge_attr.shape
    L = We.shape[0]

    src = edge_index[0]
    dst = edge_index[1]

    w2 = jnp.concatenate([We[:, :d, :], We[:, d:2 * d, :]], axis=2)
    eye8 = jnp.eye(8, dtype=jnp.float32)
    w8 = jnp.stack([jnp.kron(eye8, We[i, 2 * d:, :]) for i in range(L)])
    b8 = jnp.tile(be, (1, 8))

    batch_r = batch.reshape(n, 1)
    batch_c = batch.reshape(1, n)
    tables = []
    for i in range(L):
        s, psrc, pdst = _ln_layer(s, batch_r, batch_c, ln_gamma[i],
                                  ln_beta[i], w2[i])
        tables.append((psrc, pdst))

    e2d = edge_attr.reshape(e_cnt // 8, 8 * ed)
    for i in range(L):
        r2d = _edge_mm(e2d, w8[i], b8[i])
        e2d = _sc_edge(src, dst, tables[i][0], tables[i][1], r2d)

    return (s, e2d.reshape(e_cnt, ed))
